# baseline (speedup 1.0000x reference)
import jax
import jax.numpy as jnp
from jax.experimental import pallas as pl

B, C_IN, N_T = 4, 384, 196
K, D = 1024, 64

_BF = jnp.bfloat16
_F32 = jnp.float32


def _split3(x):
    x0 = x.astype(_BF)
    r1 = x - x0.astype(_F32)
    x1 = r1.astype(_BF)
    x2 = (r1 - x1.astype(_F32)).astype(_BF)
    return x0, x1, x2


def _vq_body(z_ref, w_ref, emb_ref, out_ref):
    w = w_ref[...]
    emb = emb_ref[...]
    emb2 = jnp.sum(emb * emb, axis=1, keepdims=True)
    emb_norm = jnp.sqrt(emb2)

    e0, e1, e2 = _split3(emb)
    m2e0, m2e1, m2e2 = (-2.0 * e0.astype(_F32)).astype(_BF), \
                       (-2.0 * e1.astype(_F32)).astype(_BF), \
                       (-2.0 * e2.astype(_F32)).astype(_BF)
    a1 = jnp.concatenate([m2e0, m2e1, m2e2, m2e0], axis=1)
    q0, q1, q2 = _split3(emb2)
    onesk = jnp.ones((K, 3), _BF)
    a2 = jnp.concatenate([m2e0, m2e1, q0.astype(_BF), q1.astype(_BF),
                          q2.astype(_BF), onesk], axis=1)

    et0, et1, _ = _split3(emb.T)
    at = jnp.concatenate([et0, et1], axis=1)

    iota_k = jax.lax.broadcasted_iota(jnp.int32, (K, N_T), 0)
    for b in range(B):
        zb = z_ref[b]
        ze = jnp.dot(w, zb)
        ze2 = jnp.sum(ze * ze, axis=0, keepdims=True)
        s0, s1, s2p = _split3(ze)
        t0, t1, t2 = _split3(ze2)
        x1 = jnp.concatenate([s0, s0, s0, s2p], axis=0)
        ones_n = jnp.ones((1, N_T), _BF)
        x2 = jnp.concatenate([s1, s1, ones_n, ones_n, ones_n,
                              t0.astype(_BF), t1.astype(_BF), t2.astype(_BF)],
                             axis=0)
        num2 = (jnp.dot(a1, x1, preferred_element_type=_F32)
                + jnp.dot(a2, x2, preferred_element_type=_F32))
        den = jnp.sqrt(ze2) + emb_norm
        s2 = num2 / (den * den)
        mins = jnp.min(s2, axis=0, keepdims=True)
        idx = jnp.min(jnp.where(s2 == mins, iota_k, K), axis=0, keepdims=True)
        onehot = (iota_k == idx).astype(_BF)
        zq = jnp.dot(at, jnp.concatenate([onehot, onehot], axis=0),
                     preferred_element_type=_F32)
        out_ref[b] = ze + (zq - ze)


@jax.jit
def kernel(z, W, emb):
    return pl.pallas_call(
        _vq_body,
        out_shape=jax.ShapeDtypeStruct((B, D, N_T), jnp.float32),
    )(z, W, emb)
